# Initial kernel scaffold; baseline (speedup 1.0000x reference)
import jax
import jax.numpy as jnp
from jax.experimental import pallas as pl


def kernel(norm, pos, x, edge_index, edge_attr, batch, W1, b1, W2, b2, W3, b3, Wl, bl):
    raise NotImplementedError("write your pallas kernel here")


# speedup vs baseline: 5.0364x; 5.0364x over previous
import functools

import jax
import jax.numpy as jnp
from jax import lax
from jax.experimental import pallas as pl
from jax.experimental.pallas import tpu as pltpu
from jax.experimental.pallas import tpu_sc as plsc

N = 10000
E = 320000
G = 8
NCLS = 10
NC = 2
NS = 16
NW = NC * NS
EPT = E // NW
K = 80
NCH = EPT // K
RPT = 624
TAIL = N - NS * RPT


def _make_segsum(d):
    mesh = plsc.VectorSubcoreMesh(core_axis_name="c", subcore_axis_name="s")

    @functools.partial(
        pl.kernel,
        out_type=jax.ShapeDtypeStruct((NC, N, d), jnp.float32),
        mesh=mesh,
        scratch_types=[
            pltpu.VMEM((K,), jnp.int32),
            pltpu.VMEM((K,), jnp.int32),
            pltpu.VMEM((K, d), jnp.float32),
            pltpu.VMEM((RPT, d), jnp.float32),
            pltpu.VMEM_SHARED((N, d), jnp.float32),
            pltpu.SemaphoreType.DMA,
        ],
        compiler_params=pltpu.CompilerParams(use_tc_tiling_on_sc=False),
    )
    def seg(h_hbm, src_hbm, dst_hbm, out_hbm, src_v, dst_v, rows_v, zer_v, acc_s, sem):
        cid = lax.axis_index("c")
        sid = lax.axis_index("s")

        def zbody(i, _):
            for c in range(d // 16):
                zer_v[i, pl.ds(c * 16, 16)] = jnp.zeros((16,), jnp.float32)
            return 0

        lax.fori_loop(0, RPT, zbody, 0)
        pltpu.sync_copy(zer_v, acc_s.at[pl.ds(sid * RPT, RPT)])

        @pl.when(sid == NS - 1)
        def _():
            pltpu.sync_copy(zer_v.at[pl.ds(0, TAIL)],
                            acc_s.at[pl.ds(NS * RPT, TAIL)])

        plsc.subcore_barrier()

        ebase = (cid * NS + sid) * EPT

        def body(j, _):
            base = ebase + j * K
            pltpu.sync_copy(src_hbm.at[pl.ds(base, K)], src_v)
            pltpu.sync_copy(dst_hbm.at[pl.ds(base, K)], dst_v)
            pltpu.async_copy(h_hbm.at[src_v], rows_v, sem).wait()
            pltpu.sync_copy(rows_v, acc_s.at[dst_v], add=True)
            return 0

        lax.fori_loop(0, NCH, body, 0)
        plsc.subcore_barrier()
        pltpu.sync_copy(acc_s.at[pl.ds(sid * RPT, RPT)],
                        out_hbm.at[cid, pl.ds(sid * RPT, RPT)])

        @pl.when(sid == NS - 1)
        def _():
            pltpu.sync_copy(acc_s.at[pl.ds(NS * RPT, TAIL)],
                            out_hbm.at[cid, pl.ds(NS * RPT, TAIL)])

    return seg


_segsum16 = _make_segsum(16)
_segsum64 = _make_segsum(64)


R_BLK = 2000
_GRID = N // R_BLK


def _full(shape):
    return pl.BlockSpec(shape, lambda i: (0,) * len(shape))


def _rows(w):
    return pl.BlockSpec((R_BLK, w), lambda i: (i, 0))


def _mm1_body(s1a, s1b, w, b, o):
    s = s1a[...] + s1b[...]
    o[...] = jnp.maximum(
        jnp.dot(s, w[...], preferred_element_type=jnp.float32) + b[...], 0.0)


def _mm1(s1a, s1b, w, b):
    return pl.pallas_call(
        _mm1_body,
        grid=(_GRID,),
        in_specs=[_rows(16), _rows(16), _full(w.shape), _full(b.shape)],
        out_specs=_rows(64),
        out_shape=jax.ShapeDtypeStruct((N, 64), jnp.float32),
    )(s1a, s1b, w, b)


def _mm2_body(t1a, t1b, s1a, s1b, wa, wb, b, oa, ob):
    t1 = t1a[...] + t1b[...]
    s1 = s1a[...] + s1b[...]
    acc = jnp.dot(t1, wa[...], preferred_element_type=jnp.float32)
    acc += jnp.dot(s1, wb[...], preferred_element_type=jnp.float32)
    h2 = jnp.maximum(acc + b[...], 0.0)
    oa[...] = h2[:, :64]
    ob[...] = h2[:, 64:]


def _mm2(t1a, t1b, s1a, s1b, wa, wb, b):
    return pl.pallas_call(
        _mm2_body,
        grid=(_GRID,),
        in_specs=[_rows(64), _rows(64), _rows(16), _rows(16),
                  _full(wa.shape), _full(wb.shape), _full(b.shape)],
        out_specs=[_rows(64), _rows(64)],
        out_shape=[jax.ShapeDtypeStruct((N, 64), jnp.float32),
                   jax.ShapeDtypeStruct((N, 64), jnp.float32)],
    )(t1a, t1b, s1a, s1b, wa, wb, b)


def _mm3pool_body(t1a, t1b, s1a, s1b, t2aa, t2ab, t2ba, t2bb, bat,
                  wa, wb, wct, wcb, b, o):
    t1 = t1a[...] + t1b[...]
    s1 = s1a[...] + s1b[...]
    t2a = t2aa[...] + t2ab[...]
    t2b = t2ba[...] + t2bb[...]
    acc = jnp.dot(t1, wa[...], preferred_element_type=jnp.float32)
    acc += jnp.dot(s1, wb[...], preferred_element_type=jnp.float32)
    acc += jnp.dot(t2a, wct[...], preferred_element_type=jnp.float32)
    acc += jnp.dot(t2b, wcb[...], preferred_element_type=jnp.float32)
    h3 = jnp.maximum(acc + b[...], 0.0)
    neg = jnp.float32(-jnp.inf)
    bt = bat[...]
    pooled = jnp.stack(
        [jnp.max(jnp.where(bt == g, h3, neg), axis=0) for g in range(G)], axis=0)

    @pl.when(pl.program_id(0) == 0)
    def _():
        o[...] = jnp.full((G, 256), neg, jnp.float32)

    o[...] = jnp.maximum(o[...], pooled)


def _mm3pool(t1a, t1b, s1a, s1b, t2aa, t2ab, t2ba, t2bb, bat, wa, wb, wct, wcb, b):
    return pl.pallas_call(
        _mm3pool_body,
        grid=(_GRID,),
        in_specs=[_rows(64), _rows(64), _rows(16), _rows(16),
                  _rows(64), _rows(64), _rows(64), _rows(64), _rows(1),
                  _full(wa.shape), _full(wb.shape), _full(wct.shape),
                  _full(wcb.shape), _full(b.shape)],
        out_specs=pl.BlockSpec((G, 256), lambda i: (0, 0)),
        out_shape=jax.ShapeDtypeStruct((G, 256), jnp.float32),
    )(t1a, t1b, s1a, s1b, t2aa, t2ab, t2ba, t2bb, bat, wa, wb, wct, wcb, b)


def _head_body(p, wl, bl, out_r, pred_r):
    logits = jnp.dot(p[...], wl[...], preferred_element_type=jnp.float32) + bl[...]
    m = jnp.max(logits, axis=1, keepdims=True)
    e = jnp.exp(logits - m)
    lse = jnp.log(jnp.sum(e, axis=1, keepdims=True)) + m
    out = logits - lse
    out_r[...] = out
    pred_r[...] = jnp.exp(out)


def _head(p, wl, bl):
    return pl.pallas_call(
        _head_body,
        grid=(1,),
        in_specs=[_full(p.shape), _full(wl.shape), _full(bl.shape)],
        out_specs=[_full((G, NCLS)), _full((G, NCLS))],
        out_shape=[jax.ShapeDtypeStruct((G, NCLS), jnp.float32),
                   jax.ShapeDtypeStruct((G, NCLS), jnp.float32)],
    )(p, wl, bl)


@jax.jit
def kernel(norm, pos, x, edge_index, edge_attr, batch, W1, b1, W2, b2, W3, b3, Wl, bl):
    inp16 = jnp.concatenate(
        [norm, pos, x, jnp.zeros((N, 8), jnp.float32)], axis=1)
    src = edge_index[0]
    dst = edge_index[1]

    W1p = jnp.pad(W1, ((0, 8), (0, 0)))
    W2a = W2[:64]
    W2b = jnp.pad(W2[64:72], ((0, 8), (0, 0)))
    W3a = W3[:64]
    W3bd = jnp.pad(W3[64:72] + W3[200:208], ((0, 8), (0, 0)))
    W3ct = W3[72:136]
    W3cb = W3[136:200]

    S1 = _segsum16(inp16, src, dst)
    h1 = _mm1(S1[0], S1[1], W1p, b1[None, :])
    T1 = _segsum64(h1, src, dst)
    h2a, h2b = _mm2(T1[0], T1[1], S1[0], S1[1], W2a, W2b, b2[None, :])
    T2a = _segsum64(h2a, src, dst)
    T2b = _segsum64(h2b, src, dst)
    pooled = _mm3pool(T1[0], T1[1], S1[0], S1[1],
                      T2a[0], T2a[1], T2b[0], T2b[1],
                      batch[:, None], W3a, W3bd, W3ct, W3cb, b3[None, :])
    out, pred = _head(pooled, Wl, bl[None, :])
    return (out, pred)


# speedup vs baseline: 16.1026x; 3.1973x over previous
import functools

import jax
import jax.numpy as jnp
from jax import lax
from jax.experimental import pallas as pl
from jax.experimental.pallas import tpu as pltpu
from jax.experimental.pallas import tpu_sc as plsc

N = 10000
E = 320000
G = 8
NCLS = 10
NC = 2
NS = 16
NW = NC * NS
EPT = E // NW
K = 80
NCH = EPT // K
RPT = 624
TAIL = N - NS * RPT


NBUF = 4
assert (NCH - 1) % NBUF == 0


def _make_segsum(d):
    mesh = plsc.VectorSubcoreMesh(core_axis_name="c", subcore_axis_name="s")

    @functools.partial(
        pl.kernel,
        out_type=jax.ShapeDtypeStruct((NC, N, d), jnp.float32),
        mesh=mesh,
        scratch_types=[
            pltpu.VMEM((NCH, K), jnp.int32),
            pltpu.VMEM((NCH, K), jnp.int32),
            pltpu.VMEM((NBUF, K, d), jnp.float32),
            pltpu.VMEM((RPT, d), jnp.float32),
            pltpu.VMEM_SHARED((N, d), jnp.float32),
            pltpu.SemaphoreType.DMA((NBUF,)),
        ],
        compiler_params=pltpu.CompilerParams(use_tc_tiling_on_sc=False),
    )
    def seg(h_hbm, src_hbm, dst_hbm, out_hbm, src_v, dst_v, rows_v, zer_v, acc_s, gsem):
        cid = lax.axis_index("c")
        sid = lax.axis_index("s")
        wid = cid * NS + sid

        pltpu.sync_copy(src_hbm.at[wid], src_v)
        pltpu.sync_copy(dst_hbm.at[wid], dst_v)

        def zbody(i, _):
            for c in range(d // 16):
                zer_v[i, pl.ds(c * 16, 16)] = jnp.zeros((16,), jnp.float32)
            return 0

        lax.fori_loop(0, RPT, zbody, 0)
        pltpu.sync_copy(zer_v, acc_s.at[pl.ds(sid * RPT, RPT)])

        @pl.when(sid == NS - 1)
        def _():
            pltpu.sync_copy(zer_v.at[pl.ds(0, TAIL)],
                            acc_s.at[pl.ds(NS * RPT, TAIL)])

        plsc.subcore_barrier()

        for b in range(NBUF):
            pltpu.async_copy(h_hbm.at[src_v.at[b]], rows_v.at[b], gsem.at[b])

        def body(i, _):
            for b in range(NBUF):
                c = i * NBUF + b
                pltpu.make_async_copy(h_hbm.at[src_v.at[c]], rows_v.at[b],
                                      gsem.at[b]).wait()
                pltpu.sync_copy(rows_v.at[b], acc_s.at[dst_v.at[c]], add=True)
                nxt = c + NBUF

                @pl.when(nxt < NCH)
                def _():
                    pltpu.async_copy(h_hbm.at[src_v.at[nxt]], rows_v.at[b],
                                     gsem.at[b])
            return 0

        lax.fori_loop(0, (NCH - 1) // NBUF, body, 0)
        lb = (NCH - 1) % NBUF
        pltpu.make_async_copy(h_hbm.at[src_v.at[NCH - 1]], rows_v.at[lb],
                              gsem.at[lb]).wait()
        pltpu.sync_copy(rows_v.at[lb], acc_s.at[dst_v.at[NCH - 1]], add=True)
        plsc.subcore_barrier()
        pltpu.sync_copy(acc_s.at[pl.ds(sid * RPT, RPT)],
                        out_hbm.at[cid, pl.ds(sid * RPT, RPT)])

        @pl.when(sid == NS - 1)
        def _():
            pltpu.sync_copy(acc_s.at[pl.ds(NS * RPT, TAIL)],
                            out_hbm.at[cid, pl.ds(NS * RPT, TAIL)])

    return seg


_segsum16 = _make_segsum(16)
_segsum64 = _make_segsum(64)


R_BLK = 2000
_GRID = N // R_BLK


def _full(shape):
    return pl.BlockSpec(shape, lambda i: (0,) * len(shape))


def _rows(w):
    return pl.BlockSpec((R_BLK, w), lambda i: (i, 0))


def _mm1_body(s1a, s1b, w, b, o):
    s = s1a[...] + s1b[...]
    o[...] = jnp.maximum(
        jnp.dot(s, w[...], preferred_element_type=jnp.float32) + b[...], 0.0)


def _mm1(s1a, s1b, w, b):
    return pl.pallas_call(
        _mm1_body,
        grid=(_GRID,),
        in_specs=[_rows(16), _rows(16), _full(w.shape), _full(b.shape)],
        out_specs=_rows(64),
        out_shape=jax.ShapeDtypeStruct((N, 64), jnp.float32),
    )(s1a, s1b, w, b)


def _mm2_body(t1a, t1b, s1a, s1b, wa, wb, b, oa, ob):
    t1 = t1a[...] + t1b[...]
    s1 = s1a[...] + s1b[...]
    acc = jnp.dot(t1, wa[...], preferred_element_type=jnp.float32)
    acc += jnp.dot(s1, wb[...], preferred_element_type=jnp.float32)
    h2 = jnp.maximum(acc + b[...], 0.0)
    oa[...] = h2[:, :64]
    ob[...] = h2[:, 64:]


def _mm2(t1a, t1b, s1a, s1b, wa, wb, b):
    return pl.pallas_call(
        _mm2_body,
        grid=(_GRID,),
        in_specs=[_rows(64), _rows(64), _rows(16), _rows(16),
                  _full(wa.shape), _full(wb.shape), _full(b.shape)],
        out_specs=[_rows(64), _rows(64)],
        out_shape=[jax.ShapeDtypeStruct((N, 64), jnp.float32),
                   jax.ShapeDtypeStruct((N, 64), jnp.float32)],
    )(t1a, t1b, s1a, s1b, wa, wb, b)


def _mm3pool_body(t1a, t1b, s1a, s1b, t2aa, t2ab, t2ba, t2bb, bat,
                  wa, wb, wct, wcb, b, o):
    t1 = t1a[...] + t1b[...]
    s1 = s1a[...] + s1b[...]
    t2a = t2aa[...] + t2ab[...]
    t2b = t2ba[...] + t2bb[...]
    acc = jnp.dot(t1, wa[...], preferred_element_type=jnp.float32)
    acc += jnp.dot(s1, wb[...], preferred_element_type=jnp.float32)
    acc += jnp.dot(t2a, wct[...], preferred_element_type=jnp.float32)
    acc += jnp.dot(t2b, wcb[...], preferred_element_type=jnp.float32)
    h3 = jnp.maximum(acc + b[...], 0.0)
    neg = jnp.float32(-jnp.inf)
    bt = bat[...]
    pooled = jnp.stack(
        [jnp.max(jnp.where(bt == g, h3, neg), axis=0) for g in range(G)], axis=0)

    @pl.when(pl.program_id(0) == 0)
    def _():
        o[...] = jnp.full((G, 256), neg, jnp.float32)

    o[...] = jnp.maximum(o[...], pooled)


def _mm3pool(t1a, t1b, s1a, s1b, t2aa, t2ab, t2ba, t2bb, bat, wa, wb, wct, wcb, b):
    return pl.pallas_call(
        _mm3pool_body,
        grid=(_GRID,),
        in_specs=[_rows(64), _rows(64), _rows(16), _rows(16),
                  _rows(64), _rows(64), _rows(64), _rows(64), _rows(1),
                  _full(wa.shape), _full(wb.shape), _full(wct.shape),
                  _full(wcb.shape), _full(b.shape)],
        out_specs=pl.BlockSpec((G, 256), lambda i: (0, 0)),
        out_shape=jax.ShapeDtypeStruct((G, 256), jnp.float32),
    )(t1a, t1b, s1a, s1b, t2aa, t2ab, t2ba, t2bb, bat, wa, wb, wct, wcb, b)


def _head_body(p, wl, bl, out_r, pred_r):
    logits = jnp.dot(p[...], wl[...], preferred_element_type=jnp.float32) + bl[...]
    m = jnp.max(logits, axis=1, keepdims=True)
    e = jnp.exp(logits - m)
    lse = jnp.log(jnp.sum(e, axis=1, keepdims=True)) + m
    out = logits - lse
    out_r[...] = out
    pred_r[...] = jnp.exp(out)


def _head(p, wl, bl):
    return pl.pallas_call(
        _head_body,
        grid=(1,),
        in_specs=[_full(p.shape), _full(wl.shape), _full(bl.shape)],
        out_specs=[_full((G, NCLS)), _full((G, NCLS))],
        out_shape=[jax.ShapeDtypeStruct((G, NCLS), jnp.float32),
                   jax.ShapeDtypeStruct((G, NCLS), jnp.float32)],
    )(p, wl, bl)


@jax.jit
def kernel(norm, pos, x, edge_index, edge_attr, batch, W1, b1, W2, b2, W3, b3, Wl, bl):
    inp16 = jnp.concatenate(
        [norm, pos, x, jnp.zeros((N, 8), jnp.float32)], axis=1)
    src = edge_index[0].reshape(NW, NCH, K)
    dst = edge_index[1].reshape(NW, NCH, K)

    W1p = jnp.pad(W1, ((0, 8), (0, 0)))
    W2a = W2[:64]
    W2b = jnp.pad(W2[64:72], ((0, 8), (0, 0)))
    W3a = W3[:64]
    W3bd = jnp.pad(W3[64:72] + W3[200:208], ((0, 8), (0, 0)))
    W3ct = W3[72:136]
    W3cb = W3[136:200]

    S1 = _segsum16(inp16, src, dst)
    h1 = _mm1(S1[0], S1[1], W1p, b1[None, :])
    T1 = _segsum64(h1, src, dst)
    h2a, h2b = _mm2(T1[0], T1[1], S1[0], S1[1], W2a, W2b, b2[None, :])
    T2a = _segsum64(h2a, src, dst)
    T2b = _segsum64(h2b, src, dst)
    pooled = _mm3pool(T1[0], T1[1], S1[0], S1[1],
                      T2a[0], T2a[1], T2b[0], T2b[1],
                      batch[:, None], W3a, W3bd, W3ct, W3cb, b3[None, :])
    out, pred = _head(pooled, Wl, bl[None, :])
    return (out, pred)


# speedup vs baseline: 16.8701x; 1.0477x over previous
import functools

import jax
import jax.numpy as jnp
from jax import lax
from jax.experimental import pallas as pl
from jax.experimental.pallas import tpu as pltpu
from jax.experimental.pallas import tpu_sc as plsc

N = 10000
E = 320000
G = 8
NCLS = 10
NC = 2
NS = 16
NW = NC * NS
EPT = E // NW
K = 80
NCH = EPT // K
RPT = 624
TAIL = N - NS * RPT


NBUF = 5


def _make_segsum(d, split):
    mesh = plsc.VectorSubcoreMesh(core_axis_name="c", subcore_axis_name="s")
    nch = E // (NS if split else NW) // K
    assert nch % NBUF == 0

    @functools.partial(
        pl.kernel,
        out_type=jax.ShapeDtypeStruct((NC, N, d), jnp.float32),
        mesh=mesh,
        scratch_types=[
            pltpu.VMEM((nch, K), jnp.int32),
            pltpu.VMEM((nch, K), jnp.int32),
            pltpu.VMEM((NBUF, K, d), jnp.float32),
            pltpu.VMEM((RPT, d), jnp.float32),
            pltpu.VMEM_SHARED((N, d), jnp.float32),
            pltpu.SemaphoreType.DMA((NBUF,)),
        ],
        compiler_params=pltpu.CompilerParams(use_tc_tiling_on_sc=False),
    )
    def seg(h_hbm, src_hbm, dst_hbm, out_hbm, src_v, dst_v, rows_v, zer_v, acc_s, gsem):
        cid = lax.axis_index("c")
        sid = lax.axis_index("s")
        wid = cid * NS + sid

        pltpu.sync_copy(src_hbm.at[wid], src_v)
        pltpu.sync_copy(dst_hbm.at[sid if split else wid], dst_v)

        for b in range(NBUF):
            pltpu.async_copy(h_hbm.at[src_v.at[b]], rows_v.at[b], gsem.at[b])

        def zbody(i, _):
            for c in range(d // 16):
                zer_v[i, pl.ds(c * 16, 16)] = jnp.zeros((16,), jnp.float32)
            return 0

        lax.fori_loop(0, RPT, zbody, 0)
        pltpu.sync_copy(zer_v, acc_s.at[pl.ds(sid * RPT, RPT)])

        @pl.when(sid == NS - 1)
        def _():
            pltpu.sync_copy(zer_v.at[pl.ds(0, TAIL)],
                            acc_s.at[pl.ds(NS * RPT, TAIL)])

        plsc.subcore_barrier()

        def body(i, _):
            for b in range(NBUF):
                c = i * NBUF + b
                pltpu.make_async_copy(h_hbm.at[src_v.at[c]], rows_v.at[b],
                                      gsem.at[b]).wait()
                pltpu.sync_copy(rows_v.at[b], acc_s.at[dst_v.at[c]], add=True)
                nxt = c + NBUF

                @pl.when(nxt < nch)
                def _():
                    pltpu.async_copy(h_hbm.at[src_v.at[nxt]], rows_v.at[b],
                                     gsem.at[b])
            return 0

        lax.fori_loop(0, nch // NBUF, body, 0)
        plsc.subcore_barrier()
        pltpu.sync_copy(acc_s.at[pl.ds(sid * RPT, RPT)],
                        out_hbm.at[cid, pl.ds(sid * RPT, RPT)])

        @pl.when(sid == NS - 1)
        def _():
            pltpu.sync_copy(acc_s.at[pl.ds(NS * RPT, TAIL)],
                            out_hbm.at[cid, pl.ds(NS * RPT, TAIL)])

    return seg


_segsum16 = _make_segsum(16, split=False)
_segsum64 = _make_segsum(64, split=False)


R_BLK = 2000
_GRID = N // R_BLK


def _full(shape):
    return pl.BlockSpec(shape, lambda i: (0,) * len(shape))


def _rows(w):
    return pl.BlockSpec((R_BLK, w), lambda i: (i, 0))


def _mm1_body(s1a, s1b, w, b, o):
    s = s1a[...] + s1b[...]
    o[...] = jnp.maximum(
        jnp.dot(s, w[...], preferred_element_type=jnp.float32) + b[...], 0.0)


def _mm1(s1a, s1b, w, b):
    return pl.pallas_call(
        _mm1_body,
        grid=(_GRID,),
        in_specs=[_rows(16), _rows(16), _full(w.shape), _full(b.shape)],
        out_specs=_rows(64),
        out_shape=jax.ShapeDtypeStruct((N, 64), jnp.float32),
    )(s1a, s1b, w, b)


def _mm2_body(t1a, t1b, s1a, s1b, wa, wb, b, o, o2):
    t1 = t1a[...] + t1b[...]
    s1 = s1a[...] + s1b[...]
    acc = jnp.dot(t1, wa[...], preferred_element_type=jnp.float32)
    acc += jnp.dot(s1, wb[...], preferred_element_type=jnp.float32)
    h2 = jnp.maximum(acc + b[...], 0.0)
    o[...] = h2[:, :64]
    o2[...] = h2[:, 64:]


def _mm2(t1a, t1b, s1a, s1b, wa, wb, b):
    return pl.pallas_call(
        _mm2_body,
        grid=(_GRID,),
        in_specs=[_rows(64), _rows(64), _rows(16), _rows(16),
                  _full(wa.shape), _full(wb.shape), _full(b.shape)],
        out_specs=[_rows(64), _rows(64)],
        out_shape=[jax.ShapeDtypeStruct((N, 64), jnp.float32),
                   jax.ShapeDtypeStruct((N, 64), jnp.float32)],
    )(t1a, t1b, s1a, s1b, wa, wb, b)


def _mm3pool_body(t1a, t1b, s1a, s1b, t2aa, t2ab, t2ba, t2bb, bat,
                  wa, wb, wct, wcb, b, o):
    t1 = t1a[...] + t1b[...]
    s1 = s1a[...] + s1b[...]
    t2a = t2aa[...] + t2ab[...]
    t2b = t2ba[...] + t2bb[...]
    acc = jnp.dot(t1, wa[...], preferred_element_type=jnp.float32)
    acc += jnp.dot(s1, wb[...], preferred_element_type=jnp.float32)
    acc += jnp.dot(t2a, wct[...], preferred_element_type=jnp.float32)
    acc += jnp.dot(t2b, wcb[...], preferred_element_type=jnp.float32)
    h3 = jnp.maximum(acc + b[...], 0.0)
    neg = jnp.float32(-jnp.inf)
    bt = bat[...]
    pooled = jnp.stack(
        [jnp.max(jnp.where(bt == g, h3, neg), axis=0) for g in range(G)], axis=0)

    @pl.when(pl.program_id(0) == 0)
    def _():
        o[...] = jnp.full((G, 256), neg, jnp.float32)

    o[...] = jnp.maximum(o[...], pooled)


def _mm3pool(t1a, t1b, s1a, s1b, t2aa, t2ab, t2ba, t2bb, bat, wa, wb, wct, wcb, b):
    return pl.pallas_call(
        _mm3pool_body,
        grid=(_GRID,),
        in_specs=[_rows(64), _rows(64), _rows(16), _rows(16),
                  _rows(64), _rows(64), _rows(64), _rows(64), _rows(1),
                  _full(wa.shape), _full(wb.shape), _full(wct.shape),
                  _full(wcb.shape), _full(b.shape)],
        out_specs=pl.BlockSpec((G, 256), lambda i: (0, 0)),
        out_shape=jax.ShapeDtypeStruct((G, 256), jnp.float32),
    )(t1a, t1b, s1a, s1b, t2aa, t2ab, t2ba, t2bb, bat, wa, wb, wct, wcb, b)


def _head_body(p, wl, bl, out_r, pred_r):
    logits = jnp.dot(p[...], wl[...], preferred_element_type=jnp.float32) + bl[...]
    m = jnp.max(logits, axis=1, keepdims=True)
    e = jnp.exp(logits - m)
    lse = jnp.log(jnp.sum(e, axis=1, keepdims=True)) + m
    out = logits - lse
    out_r[...] = out
    pred_r[...] = jnp.exp(out)


def _head(p, wl, bl):
    return pl.pallas_call(
        _head_body,
        grid=(1,),
        in_specs=[_full(p.shape), _full(wl.shape), _full(bl.shape)],
        out_specs=[_full((G, NCLS)), _full((G, NCLS))],
        out_shape=[jax.ShapeDtypeStruct((G, NCLS), jnp.float32),
                   jax.ShapeDtypeStruct((G, NCLS), jnp.float32)],
    )(p, wl, bl)


@jax.jit
def kernel(norm, pos, x, edge_index, edge_attr, batch, W1, b1, W2, b2, W3, b3, Wl, bl):
    inp16 = jnp.concatenate(
        [norm, pos, x, jnp.zeros((N, 8), jnp.float32)], axis=1)
    src = edge_index[0].reshape(NW, NCH, K)
    dst = edge_index[1].reshape(NW, NCH, K)

    W1p = jnp.pad(W1, ((0, 8), (0, 0)))
    W2a = W2[:64]
    W2b = jnp.pad(W2[64:72], ((0, 8), (0, 0)))
    W3a = W3[:64]
    W3bd = jnp.pad(W3[64:72] + W3[200:208], ((0, 8), (0, 0)))
    W3ct = W3[72:136]
    W3cb = W3[136:200]

    S1 = _segsum16(inp16, src, dst)
    h1 = _mm1(S1[0], S1[1], W1p, b1[None, :])
    T1 = _segsum64(h1, src, dst)
    h2a, h2b = _mm2(T1[0], T1[1], S1[0], S1[1], W2a, W2b, b2[None, :])
    T2a = _segsum64(h2a, src, dst)
    T2b = _segsum64(h2b, src, dst)
    pooled = _mm3pool(T1[0], T1[1], S1[0], S1[1],
                      T2a[0], T2a[1], T2b[0], T2b[1],
                      batch[:, None], W3a, W3bd, W3ct, W3cb, b3[None, :])
    out, pred = _head(pooled, Wl, bl[None, :])
    return (out, pred)


# speedup vs baseline: 18.5077x; 1.0971x over previous
import functools

import jax
import jax.numpy as jnp
from jax import lax
from jax.experimental import pallas as pl
from jax.experimental.pallas import tpu as pltpu
from jax.experimental.pallas import tpu_sc as plsc

N = 10000
E = 320000
G = 8
NCLS = 10
NC = 2
NS = 16
NW = NC * NS
EPT = E // NW
K = 80
NCH = EPT // K
RPT = 624
TAIL = N - NS * RPT
NBUF = 5
assert NCH % NBUF == 0


def _make_segsum(d):
    mesh = plsc.VectorSubcoreMesh(core_axis_name="c", subcore_axis_name="s")

    @functools.partial(
        pl.kernel,
        out_type=jax.ShapeDtypeStruct((NC, N, d), jnp.float32),
        mesh=mesh,
        scratch_types=[
            pltpu.VMEM((EPT,), jnp.int32),
            pltpu.VMEM((EPT,), jnp.int32),
            pltpu.VMEM((NBUF, K, d), jnp.float32),
            pltpu.VMEM((RPT, d), jnp.float32),
            pltpu.VMEM_SHARED((N, d), jnp.float32),
            pltpu.SemaphoreType.DMA((NBUF,)),
        ],
        compiler_params=pltpu.CompilerParams(use_tc_tiling_on_sc=False),
    )
    def seg(h_hbm, src_hbm, dst_hbm, out_hbm, src_v, dst_v, rows_v, zer_v, acc_s, gsem):
        cid = lax.axis_index("c")
        sid = lax.axis_index("s")
        wid = cid * NS + sid

        pltpu.sync_copy(src_hbm.at[pl.ds(wid * EPT, EPT)], src_v)
        pltpu.sync_copy(dst_hbm.at[pl.ds(wid * EPT, EPT)], dst_v)

        for b in range(NBUF):
            pltpu.async_copy(h_hbm.at[src_v.at[pl.ds(b * K, K)]], rows_v.at[b],
                             gsem.at[b])

        def zbody(i, _):
            for c in range(d // 16):
                zer_v[i, pl.ds(c * 16, 16)] = jnp.zeros((16,), jnp.float32)
            return 0

        lax.fori_loop(0, RPT, zbody, 0)
        pltpu.sync_copy(zer_v, acc_s.at[pl.ds(sid * RPT, RPT)])

        @pl.when(sid == NS - 1)
        def _():
            pltpu.sync_copy(zer_v.at[pl.ds(0, TAIL)],
                            acc_s.at[pl.ds(NS * RPT, TAIL)])

        plsc.subcore_barrier()

        def body(i, _):
            for b in range(NBUF):
                c = i * NBUF + b
                pltpu.make_async_copy(h_hbm.at[src_v.at[pl.ds(c * K, K)]],
                                      rows_v.at[b], gsem.at[b]).wait()
                pltpu.sync_copy(rows_v.at[b],
                                acc_s.at[dst_v.at[pl.ds(c * K, K)]], add=True)
                nxt = c + NBUF

                @pl.when(nxt < NCH)
                def _():
                    pltpu.async_copy(h_hbm.at[src_v.at[pl.ds(nxt * K, K)]],
                                     rows_v.at[b], gsem.at[b])
            return 0

        lax.fori_loop(0, NCH // NBUF, body, 0)
        plsc.subcore_barrier()
        pltpu.sync_copy(acc_s.at[pl.ds(sid * RPT, RPT)],
                        out_hbm.at[cid, pl.ds(sid * RPT, RPT)])

        @pl.when(sid == NS - 1)
        def _():
            pltpu.sync_copy(acc_s.at[pl.ds(NS * RPT, TAIL)],
                            out_hbm.at[cid, pl.ds(NS * RPT, TAIL)])

    return seg


_segsum16 = _make_segsum(16)
_segsum64 = _make_segsum(64)


R_BLK = 2000
_GRID = N // R_BLK


def _full(shape):
    return pl.BlockSpec(shape, lambda i: (0,) * len(shape))


def _part(w, j):
    return pl.BlockSpec((1, R_BLK, w), lambda i, j=j: (j, i, 0))


def _rows(w):
    return pl.BlockSpec((R_BLK, w), lambda i: (i, 0))


def _mm1_body(s1a, s1b, w, b, o):
    s = (s1a[0] + s1b[0])[:, :8]
    o[...] = jnp.maximum(
        jnp.dot(s, w[...], preferred_element_type=jnp.float32) + b[...], 0.0)


def _mm1(s1, w, b):
    return pl.pallas_call(
        _mm1_body,
        grid=(_GRID,),
        in_specs=[_part(16, 0), _part(16, 1), _full(w.shape), _full(b.shape)],
        out_specs=_rows(64),
        out_shape=jax.ShapeDtypeStruct((N, 64), jnp.float32),
    )(s1, s1, w, b)


def _mm2_body(t1a, t1b, s1a, s1b, w, b, o, o2):
    t1 = t1a[0] + t1b[0]
    s1 = (s1a[0] + s1b[0])[:, :8]
    acc = jnp.dot(t1, w[:64], preferred_element_type=jnp.float32)
    acc += jnp.dot(s1, w[64:72], preferred_element_type=jnp.float32)
    h2 = jnp.maximum(acc + b[...], 0.0)
    o[...] = h2[:, :64]
    o2[...] = h2[:, 64:]


def _mm2(t1, s1, w, b):
    return pl.pallas_call(
        _mm2_body,
        grid=(_GRID,),
        in_specs=[_part(64, 0), _part(64, 1), _part(16, 0), _part(16, 1),
                  _full(w.shape), _full(b.shape)],
        out_specs=[_rows(64), _rows(64)],
        out_shape=[jax.ShapeDtypeStruct((N, 64), jnp.float32),
                   jax.ShapeDtypeStruct((N, 64), jnp.float32)],
    )(t1, t1, s1, s1, w, b)


def _mm3pool_body(t1a, t1b, s1a, s1b, t2aa, t2ab, t2ba, t2bb, bat, w, b, o):
    t1 = t1a[0] + t1b[0]
    s1 = (s1a[0] + s1b[0])[:, :8]
    t2a = t2aa[0] + t2ab[0]
    t2b = t2ba[0] + t2bb[0]
    wf = w[...]
    acc = jnp.dot(t1, wf[:64], preferred_element_type=jnp.float32)
    acc += jnp.dot(s1, wf[64:72] + wf[200:208], preferred_element_type=jnp.float32)
    acc += jnp.dot(t2a, wf[72:136], preferred_element_type=jnp.float32)
    acc += jnp.dot(t2b, wf[136:200], preferred_element_type=jnp.float32)
    h3 = jnp.maximum(acc + b[...], 0.0)
    neg = jnp.float32(-jnp.inf)
    bt = bat[0, 0][:, None]
    pooled = jnp.stack(
        [jnp.max(jnp.where(bt == g, h3, neg), axis=0) for g in range(G)], axis=0)

    @pl.when(pl.program_id(0) == 0)
    def _():
        o[...] = jnp.full((G, 256), neg, jnp.float32)

    o[...] = jnp.maximum(o[...], pooled)


def _mm3pool(t1, s1, t2a, t2b, bat, w, b):
    return pl.pallas_call(
        _mm3pool_body,
        grid=(_GRID,),
        in_specs=[_part(64, 0), _part(64, 1), _part(16, 0), _part(16, 1),
                  _part(64, 0), _part(64, 1), _part(64, 0), _part(64, 1),
                  pl.BlockSpec((1, 1, R_BLK), lambda i: (i, 0, 0)),
                  _full(w.shape), _full(b.shape)],
        out_specs=pl.BlockSpec((G, 256), lambda i: (0, 0)),
        out_shape=jax.ShapeDtypeStruct((G, 256), jnp.float32),
    )(t1, t1, s1, s1, t2a, t2a, t2b, t2b, bat, w, b)


def _head_body(p, wl, bl, out_r, pred_r):
    logits = jnp.dot(p[...], wl[...], preferred_element_type=jnp.float32) + bl[...]
    m = jnp.max(logits, axis=1, keepdims=True)
    e = jnp.exp(logits - m)
    lse = jnp.log(jnp.sum(e, axis=1, keepdims=True)) + m
    out = logits - lse
    out_r[...] = out
    pred_r[...] = jnp.exp(out)


def _head(p, wl, bl):
    return pl.pallas_call(
        _head_body,
        grid=(1,),
        in_specs=[_full(p.shape), _full(wl.shape), _full(bl.shape)],
        out_specs=[_full((G, NCLS)), _full((G, NCLS))],
        out_shape=[jax.ShapeDtypeStruct((G, NCLS), jnp.float32),
                   jax.ShapeDtypeStruct((G, NCLS), jnp.float32)],
    )(p, wl, bl)


@jax.jit
def kernel(norm, pos, x, edge_index, edge_attr, batch, W1, b1, W2, b2, W3, b3, Wl, bl):
    inp16 = jnp.concatenate(
        [norm, pos, x, jnp.zeros((N, 8), jnp.float32)], axis=1)
    src = edge_index[0]
    dst = edge_index[1]

    S1 = _segsum16(inp16, src, dst)
    h1 = _mm1(S1, W1, b1[None, :])
    T1 = _segsum64(h1, src, dst)
    h2a, h2b = _mm2(T1, S1, W2, b2[None, :])
    T2a = _segsum64(h2a, src, dst)
    T2b = _segsum64(h2b, src, dst)
    pooled = _mm3pool(T1, S1, T2a, T2b, batch.reshape(_GRID, 1, R_BLK),
                      W3, b3[None, :])
    out, pred = _head(pooled, Wl, bl[None, :])
    return (out, pred)
